# Initial kernel scaffold; baseline (speedup 1.0000x reference)
import functools

import jax
import jax.numpy as jnp
from jax import lax
from jax.experimental import pallas as pl
from jax.experimental.pallas import tpu as pltpu
from jax.experimental.pallas import tpu_sc as plsc

N = 10000
NPAD = 10240
E = 320000
CHUNK = 128
NCHUNKS = 2528
CH_PER_TILE = NCHUNKS // 32
EPAD = NCHUNKS * CHUNK
HEADS = 8
BASES = 4
F_H = 16
F_B = BASES * F_H
ROWS_PER_TILE = NPAD // 16
BLK = 256
GRID = NPAD // BLK

_mesh = plsc.VectorSubcoreMesh(core_axis_name="c", subcore_axis_name="s")


@functools.partial(
    pl.kernel,
    out_type=jax.ShapeDtypeStruct((2, NPAD, 16), jnp.float32),
    mesh=_mesh,
    scratch_types=[
        pltpu.VMEM((CH_PER_TILE, CHUNK), jnp.int32),
        pltpu.VMEM((CHUNK, 16), jnp.float32),
        pltpu.VMEM_SHARED((NPAD, 16), jnp.float32),
    ],
)
def _sc_degree(col_hbm, out_hbm, col_v, ones_v, deg_sh):
    cid = lax.axis_index("c")
    sid = lax.axis_index("s")
    wid = sid * 2 + cid

    z16 = jnp.zeros((16,), jnp.float32)

    @pl.loop(0, CHUNK)
    def _(i):
        ones_v[i, pl.ds(0, 16)] = z16

    @pl.loop(0, ROWS_PER_TILE // CHUNK)
    def _(k):
        pltpu.sync_copy(ones_v, deg_sh.at[pl.ds(sid * ROWS_PER_TILE + k * CHUNK, CHUNK)])

    o16 = jnp.ones((16,), jnp.float32)

    @pl.loop(0, CHUNK)
    def _(i):
        ones_v[i, pl.ds(0, 16)] = o16

    pltpu.sync_copy(col_hbm.at[pl.ds(wid * CH_PER_TILE, CH_PER_TILE)], col_v)
    plsc.subcore_barrier()

    @pl.loop(0, CH_PER_TILE)
    def _(j):
        pltpu.sync_copy(ones_v, deg_sh.at[col_v.at[j]], add=True)

    plsc.subcore_barrier()
    pltpu.sync_copy(
        deg_sh.at[pl.ds(sid * ROWS_PER_TILE, ROWS_PER_TILE)],
        out_hbm.at[cid, pl.ds(sid * ROWS_PER_TILE, ROWS_PER_TILE)],
    )


@functools.partial(
    pl.kernel,
    out_type=jax.ShapeDtypeStruct((2, NPAD, F_B), jnp.float32),
    mesh=_mesh,
    scratch_types=[
        pltpu.VMEM((CH_PER_TILE, CHUNK), jnp.int32),
        pltpu.VMEM((CH_PER_TILE, CHUNK), jnp.int32),
        pltpu.VMEM((CHUNK, F_B), jnp.float32),
        pltpu.VMEM_SHARED((NPAD, F_B), jnp.float32),
    ],
)
def _sc_agg(b2_hbm, row_hbm, col_hbm, out_hbm, row_v, col_v, rows_v, agg_sh):
    cid = lax.axis_index("c")
    sid = lax.axis_index("s")
    wid = sid * 2 + cid

    z16 = jnp.zeros((16,), jnp.float32)

    @pl.loop(0, CHUNK)
    def _(i):
        @pl.loop(0, F_B // 16)
        def _(k):
            rows_v[i, pl.ds(k * 16, 16)] = z16

    @pl.loop(0, ROWS_PER_TILE // CHUNK)
    def _(k):
        pltpu.sync_copy(rows_v, agg_sh.at[pl.ds(sid * ROWS_PER_TILE + k * CHUNK, CHUNK)])

    pltpu.sync_copy(row_hbm.at[pl.ds(wid * CH_PER_TILE, CH_PER_TILE)], row_v)
    pltpu.sync_copy(col_hbm.at[pl.ds(wid * CH_PER_TILE, CH_PER_TILE)], col_v)
    plsc.subcore_barrier()

    @pl.loop(0, CH_PER_TILE)
    def _(j):
        pltpu.sync_copy(b2_hbm.at[row_v.at[j]], rows_v)
        pltpu.sync_copy(rows_v, agg_sh.at[col_v.at[j]], add=True)

    plsc.subcore_barrier()
    pltpu.sync_copy(
        agg_sh.at[pl.ds(sid * ROWS_PER_TILE, ROWS_PER_TILE)],
        out_hbm.at[cid, pl.ds(sid * ROWS_PER_TILE, ROWS_PER_TILE)],
    )


def _dense_body(x_ref, wb_ref, wc_ref, bc_ref, wr_ref, br_ref, b_ref, wt_ref, r_ref):
    xb = x_ref[...]
    b_ref[...] = jnp.dot(xb, wb_ref[...], preferred_element_type=jnp.float32)
    wt_ref[...] = jnp.dot(xb, wc_ref[...], preferred_element_type=jnp.float32) + bc_ref[...]
    r_ref[...] = jnp.dot(xb, wr_ref[...], preferred_element_type=jnp.float32) + br_ref[...]


_dense = pl.pallas_call(
    _dense_body,
    grid=(GRID,),
    in_specs=[
        pl.BlockSpec((BLK, 128), lambda i: (i, 0)),
        pl.BlockSpec((128, F_B), lambda i: (0, 0)),
        pl.BlockSpec((128, HEADS * BASES), lambda i: (0, 0)),
        pl.BlockSpec((1, HEADS * BASES), lambda i: (0, 0)),
        pl.BlockSpec((128, 128), lambda i: (0, 0)),
        pl.BlockSpec((1, 128), lambda i: (0, 0)),
    ],
    out_specs=[
        pl.BlockSpec((BLK, F_B), lambda i: (i, 0)),
        pl.BlockSpec((BLK, HEADS * BASES), lambda i: (i, 0)),
        pl.BlockSpec((BLK, 128), lambda i: (i, 0)),
    ],
    out_shape=[
        jax.ShapeDtypeStruct((NPAD, F_B), jnp.float32),
        jax.ShapeDtypeStruct((NPAD, HEADS * BASES), jnp.float32),
        jax.ShapeDtypeStruct((NPAD, 128), jnp.float32),
    ],
)


def _scale_body(d0_ref, d1_ref, bases_ref, b2_ref, dis_ref):
    deg = d0_ref[:, 0:1] + d1_ref[:, 0:1] + 1.0
    dis = lax.rsqrt(deg)
    dis_ref[...] = dis
    b2_ref[...] = bases_ref[...] * dis


_scale = pl.pallas_call(
    _scale_body,
    grid=(GRID,),
    in_specs=[
        pl.BlockSpec((BLK, 16), lambda i: (i, 0)),
        pl.BlockSpec((BLK, 16), lambda i: (i, 0)),
        pl.BlockSpec((BLK, F_B), lambda i: (i, 0)),
    ],
    out_specs=[
        pl.BlockSpec((BLK, F_B), lambda i: (i, 0)),
        pl.BlockSpec((BLK, 1), lambda i: (i, 0)),
    ],
    out_shape=[
        jax.ShapeDtypeStruct((NPAD, F_B), jnp.float32),
        jax.ShapeDtypeStruct((NPAD, 1), jnp.float32),
    ],
)


def _finish_body(a0_ref, a1_ref, dis_ref, bases_ref, wt_ref, res_ref, bc_ref,
                 g_ref, bt_ref, o_ref):
    dis = dis_ref[...]
    aggf = dis * (a0_ref[...] + a1_ref[...]) + (dis * dis) * bases_ref[...]
    wt = wt_ref[...]
    parts = []
    for h in range(HEADS):
        acc = wt[:, h * BASES:h * BASES + 1] * aggf[:, 0:F_H]
        for b in range(1, BASES):
            acc = acc + wt[:, h * BASES + b:h * BASES + b + 1] * aggf[:, b * F_H:(b + 1) * F_H]
        parts.append(acc)
    o = jnp.concatenate(parts, axis=1) + bc_ref[...] + res_ref[...]
    mu = jnp.mean(o, axis=1, keepdims=True)
    var = jnp.mean((o - mu) * (o - mu), axis=1, keepdims=True)
    o = (o - mu) * lax.rsqrt(var + 1e-5) * g_ref[...] + bt_ref[...]
    o_ref[...] = jnp.maximum(o, 0.0)


_finish = pl.pallas_call(
    _finish_body,
    grid=(GRID,),
    in_specs=[
        pl.BlockSpec((BLK, F_B), lambda i: (i, 0)),
        pl.BlockSpec((BLK, F_B), lambda i: (i, 0)),
        pl.BlockSpec((BLK, 1), lambda i: (i, 0)),
        pl.BlockSpec((BLK, F_B), lambda i: (i, 0)),
        pl.BlockSpec((BLK, HEADS * BASES), lambda i: (i, 0)),
        pl.BlockSpec((BLK, 128), lambda i: (i, 0)),
        pl.BlockSpec((1, 128), lambda i: (0, 0)),
        pl.BlockSpec((1, 128), lambda i: (0, 0)),
        pl.BlockSpec((1, 128), lambda i: (0, 0)),
    ],
    out_specs=pl.BlockSpec((BLK, 128), lambda i: (i, 0)),
    out_shape=jax.ShapeDtypeStruct((NPAD, 128), jnp.float32),
)


def kernel(x, edge_index, W_bases, W_comb, b_comb, bias_conv, W_res, b_res,
           ln_gamma, ln_beta):
    x_pad = jnp.zeros((NPAD, 128), jnp.float32).at[:N].set(x)
    row = edge_index[0]
    col = edge_index[1]
    pad = jnp.full((EPAD - E,), N, jnp.int32)
    row_p = jnp.concatenate([row, pad]).reshape(NCHUNKS, CHUNK)
    col_p = jnp.concatenate([col, pad]).reshape(NCHUNKS, CHUNK)

    bases, wt, res = _dense(x_pad, W_bases, W_comb, b_comb.reshape(1, -1),
                            W_res, b_res.reshape(1, -1))
    degp = _sc_degree(col_p)
    b2, dis = _scale(degp[0], degp[1], bases)
    aggp = _sc_agg(b2, row_p, col_p)
    out = _finish(aggp[0], aggp[1], dis, bases, wt, res,
                  bias_conv.reshape(1, -1), ln_gamma.reshape(1, -1),
                  ln_beta.reshape(1, -1))
    return out[:N]


# speedup vs baseline: 14.0012x; 14.0012x over previous
import functools

import jax
import jax.numpy as jnp
from jax import lax
from jax.experimental import pallas as pl
from jax.experimental.pallas import tpu as pltpu
from jax.experimental.pallas import tpu_sc as plsc

N = 10000
NPAD = 10240
E = 320000
CHUNK = 128
NCHUNKS = 2560
CH_PER_TILE = NCHUNKS // 32
EPAD = NCHUNKS * CHUNK
HEADS = 8
BASES = 4
F_H = 16
F_B = BASES * F_H
ROWS_PER_TILE = NPAD // 16
BLK = 256
GRID = NPAD // BLK

_mesh = plsc.VectorSubcoreMesh(core_axis_name="c", subcore_axis_name="s")
_sc_params = pltpu.CompilerParams(use_tc_tiling_on_sc=False)


@functools.partial(
    pl.kernel,
    out_type=jax.ShapeDtypeStruct((2, NPAD, 16), jnp.float32),
    mesh=_mesh,
    compiler_params=_sc_params,
    scratch_types=[
        pltpu.VMEM((CH_PER_TILE, CHUNK), jnp.int32),
        pltpu.VMEM((CHUNK, 16), jnp.float32),
        pltpu.VMEM_SHARED((NPAD, 16), jnp.float32),
    ],
)
def _sc_degree(col_hbm, out_hbm, col_v, ones_v, deg_sh):
    cid = lax.axis_index("c")
    sid = lax.axis_index("s")
    wid = sid * 2 + cid

    z16 = jnp.zeros((16,), jnp.float32)

    @pl.loop(0, CHUNK)
    def _(i):
        ones_v[i, pl.ds(0, 16)] = z16

    @pl.loop(0, ROWS_PER_TILE // CHUNK)
    def _(k):
        pltpu.sync_copy(ones_v, deg_sh.at[pl.ds(sid * ROWS_PER_TILE + k * CHUNK, CHUNK)])

    o16 = jnp.ones((16,), jnp.float32)

    @pl.loop(0, CHUNK)
    def _(i):
        ones_v[i, pl.ds(0, 16)] = o16

    pltpu.sync_copy(col_hbm.at[pl.ds(wid * CH_PER_TILE, CH_PER_TILE)], col_v)
    plsc.subcore_barrier()

    @pl.loop(0, CH_PER_TILE)
    def _(j):
        pltpu.sync_copy(ones_v, deg_sh.at[col_v.at[j]], add=True)

    plsc.subcore_barrier()
    pltpu.sync_copy(
        deg_sh.at[pl.ds(sid * ROWS_PER_TILE, ROWS_PER_TILE)],
        out_hbm.at[cid, pl.ds(sid * ROWS_PER_TILE, ROWS_PER_TILE)],
    )


@functools.partial(
    pl.kernel,
    out_type=jax.ShapeDtypeStruct((2, NPAD, F_B), jnp.float32),
    mesh=_mesh,
    compiler_params=_sc_params,
    scratch_types=[
        pltpu.VMEM((CH_PER_TILE, CHUNK), jnp.int32),
        pltpu.VMEM((CH_PER_TILE, CHUNK), jnp.int32),
        pltpu.VMEM((CHUNK, F_B), jnp.float32),
        pltpu.VMEM_SHARED((NPAD, F_B), jnp.float32),
    ],
)
def _sc_agg(b2_hbm, row_hbm, col_hbm, out_hbm, row_v, col_v, rows_v, agg_sh):
    cid = lax.axis_index("c")
    sid = lax.axis_index("s")
    wid = sid * 2 + cid

    z16 = jnp.zeros((16,), jnp.float32)

    @pl.loop(0, CHUNK)
    def _(i):
        @pl.loop(0, F_B // 16)
        def _(k):
            rows_v[i, pl.ds(k * 16, 16)] = z16

    @pl.loop(0, ROWS_PER_TILE // CHUNK)
    def _(k):
        pltpu.sync_copy(rows_v, agg_sh.at[pl.ds(sid * ROWS_PER_TILE + k * CHUNK, CHUNK)])

    pltpu.sync_copy(row_hbm.at[pl.ds(wid * CH_PER_TILE, CH_PER_TILE)], row_v)
    pltpu.sync_copy(col_hbm.at[pl.ds(wid * CH_PER_TILE, CH_PER_TILE)], col_v)
    plsc.subcore_barrier()

    @pl.loop(0, CH_PER_TILE)
    def _(j):
        pltpu.sync_copy(b2_hbm.at[row_v.at[j]], rows_v)
        pltpu.sync_copy(rows_v, agg_sh.at[col_v.at[j]], add=True)

    plsc.subcore_barrier()
    pltpu.sync_copy(
        agg_sh.at[pl.ds(sid * ROWS_PER_TILE, ROWS_PER_TILE)],
        out_hbm.at[cid, pl.ds(sid * ROWS_PER_TILE, ROWS_PER_TILE)],
    )


def _dense_body(x_ref, wb_ref, wc_ref, bc_ref, wr_ref, br_ref, b_ref, wt_ref, r_ref):
    xb = x_ref[...]
    b_ref[...] = jnp.dot(xb, wb_ref[...], preferred_element_type=jnp.float32)
    wt_ref[...] = jnp.dot(xb, wc_ref[...], preferred_element_type=jnp.float32) + bc_ref[...]
    r_ref[...] = jnp.dot(xb, wr_ref[...], preferred_element_type=jnp.float32) + br_ref[...]


_dense = pl.pallas_call(
    _dense_body,
    grid=(GRID,),
    in_specs=[
        pl.BlockSpec((BLK, 128), lambda i: (i, 0)),
        pl.BlockSpec((128, F_B), lambda i: (0, 0)),
        pl.BlockSpec((128, HEADS * BASES), lambda i: (0, 0)),
        pl.BlockSpec((1, HEADS * BASES), lambda i: (0, 0)),
        pl.BlockSpec((128, 128), lambda i: (0, 0)),
        pl.BlockSpec((1, 128), lambda i: (0, 0)),
    ],
    out_specs=[
        pl.BlockSpec((BLK, F_B), lambda i: (i, 0)),
        pl.BlockSpec((BLK, HEADS * BASES), lambda i: (i, 0)),
        pl.BlockSpec((BLK, 128), lambda i: (i, 0)),
    ],
    out_shape=[
        jax.ShapeDtypeStruct((NPAD, F_B), jnp.float32),
        jax.ShapeDtypeStruct((NPAD, HEADS * BASES), jnp.float32),
        jax.ShapeDtypeStruct((NPAD, 128), jnp.float32),
    ],
)


def _scale_body(d0_ref, d1_ref, bases_ref, b2_ref, dis_ref):
    deg = d0_ref[:, 0:1] + d1_ref[:, 0:1] + 1.0
    dis = lax.rsqrt(deg)
    dis_ref[...] = dis
    b2_ref[...] = bases_ref[...] * dis


_scale = pl.pallas_call(
    _scale_body,
    grid=(GRID,),
    in_specs=[
        pl.BlockSpec((BLK, 16), lambda i: (i, 0)),
        pl.BlockSpec((BLK, 16), lambda i: (i, 0)),
        pl.BlockSpec((BLK, F_B), lambda i: (i, 0)),
    ],
    out_specs=[
        pl.BlockSpec((BLK, F_B), lambda i: (i, 0)),
        pl.BlockSpec((BLK, 1), lambda i: (i, 0)),
    ],
    out_shape=[
        jax.ShapeDtypeStruct((NPAD, F_B), jnp.float32),
        jax.ShapeDtypeStruct((NPAD, 1), jnp.float32),
    ],
)


def _finish_body(a0_ref, a1_ref, dis_ref, bases_ref, wt_ref, res_ref, bc_ref,
                 g_ref, bt_ref, o_ref):
    dis = dis_ref[...]
    aggf = dis * (a0_ref[...] + a1_ref[...]) + (dis * dis) * bases_ref[...]
    wt = wt_ref[...]
    parts = []
    for h in range(HEADS):
        acc = wt[:, h * BASES:h * BASES + 1] * aggf[:, 0:F_H]
        for b in range(1, BASES):
            acc = acc + wt[:, h * BASES + b:h * BASES + b + 1] * aggf[:, b * F_H:(b + 1) * F_H]
        parts.append(acc)
    o = jnp.concatenate(parts, axis=1) + bc_ref[...] + res_ref[...]
    mu = jnp.mean(o, axis=1, keepdims=True)
    var = jnp.mean((o - mu) * (o - mu), axis=1, keepdims=True)
    o = (o - mu) * lax.rsqrt(var + 1e-5) * g_ref[...] + bt_ref[...]
    o_ref[...] = jnp.maximum(o, 0.0)


_finish = pl.pallas_call(
    _finish_body,
    grid=(GRID,),
    in_specs=[
        pl.BlockSpec((BLK, F_B), lambda i: (i, 0)),
        pl.BlockSpec((BLK, F_B), lambda i: (i, 0)),
        pl.BlockSpec((BLK, 1), lambda i: (i, 0)),
        pl.BlockSpec((BLK, F_B), lambda i: (i, 0)),
        pl.BlockSpec((BLK, HEADS * BASES), lambda i: (i, 0)),
        pl.BlockSpec((BLK, 128), lambda i: (i, 0)),
        pl.BlockSpec((1, 128), lambda i: (0, 0)),
        pl.BlockSpec((1, 128), lambda i: (0, 0)),
        pl.BlockSpec((1, 128), lambda i: (0, 0)),
    ],
    out_specs=pl.BlockSpec((BLK, 128), lambda i: (i, 0)),
    out_shape=jax.ShapeDtypeStruct((NPAD, 128), jnp.float32),
)


def kernel(x, edge_index, W_bases, W_comb, b_comb, bias_conv, W_res, b_res,
           ln_gamma, ln_beta):
    x_pad = jnp.zeros((NPAD, 128), jnp.float32).at[:N].set(x)
    row = edge_index[0]
    col = edge_index[1]
    pad = jnp.full((EPAD - E,), N, jnp.int32)
    row_p = jnp.concatenate([row, pad]).reshape(NCHUNKS, CHUNK)
    col_p = jnp.concatenate([col, pad]).reshape(NCHUNKS, CHUNK)

    bases, wt, res = _dense(x_pad, W_bases, W_comb, b_comb.reshape(1, -1),
                            W_res, b_res.reshape(1, -1))
    degp = _sc_degree(col_p)
    b2, dis = _scale(degp[0], degp[1], bases)
    aggp = _sc_agg(b2, row_p, col_p)
    out = _finish(aggp[0], aggp[1], dis, bases, wt, res,
                  bias_conv.reshape(1, -1), ln_gamma.reshape(1, -1),
                  ln_beta.reshape(1, -1))
    return out[:N]


# speedup vs baseline: 14.2919x; 1.0208x over previous
import functools

import jax
import jax.numpy as jnp
import numpy as np
from jax import lax
from jax.experimental import pallas as pl
from jax.experimental.pallas import tpu as pltpu
from jax.experimental.pallas import tpu_sc as plsc

N = 10000
NPAD = 10240
E = 320000
CHUNK = 128
NCHUNKS = 2560
CH_PER_TILE = NCHUNKS // 32
EPAD = NCHUNKS * CHUNK
HEADS = 8
BASES = 4
F_H = 16
F_B = BASES * F_H
ROWS_PER_TILE = NPAD // 16
BLK = 256
GRID = NPAD // BLK

_mesh = plsc.VectorSubcoreMesh(core_axis_name="c", subcore_axis_name="s")
_sc_params = pltpu.CompilerParams(use_tc_tiling_on_sc=False)


@functools.partial(
    pl.kernel,
    out_type=jax.ShapeDtypeStruct((2, NPAD, 16), jnp.float32),
    mesh=_mesh,
    compiler_params=_sc_params,
    scratch_types=[
        pltpu.VMEM((CH_PER_TILE, CHUNK), jnp.int32),
        pltpu.VMEM((CHUNK, 16), jnp.float32),
        pltpu.VMEM_SHARED((NPAD, 16), jnp.float32),
        pltpu.SemaphoreType.DMA,
    ],
)
def _sc_degree(col_hbm, out_hbm, col_v, ones_v, deg_sh, sem):
    cid = lax.axis_index("c")
    sid = lax.axis_index("s")
    wid = sid * 2 + cid

    z16 = jnp.zeros((16,), jnp.float32)

    @pl.loop(0, CHUNK)
    def _(i):
        ones_v[i, pl.ds(0, 16)] = z16

    @pl.loop(0, ROWS_PER_TILE // CHUNK)
    def _(k):
        pltpu.sync_copy(ones_v, deg_sh.at[pl.ds(sid * ROWS_PER_TILE + k * CHUNK, CHUNK)])

    o16 = jnp.ones((16,), jnp.float32)

    @pl.loop(0, CHUNK)
    def _(i):
        ones_v[i, pl.ds(0, 16)] = o16

    pltpu.sync_copy(col_hbm.at[pl.ds(wid * CH_PER_TILE, CH_PER_TILE)], col_v)
    plsc.subcore_barrier()

    @pl.loop(0, CH_PER_TILE)
    def _(j):
        pltpu.async_copy(ones_v, deg_sh.at[col_v.at[j]], sem, add=True)

    @pl.loop(0, CH_PER_TILE)
    def _(j):
        pltpu.make_async_copy(ones_v, deg_sh.at[col_v.at[j]], sem).wait()

    plsc.subcore_barrier()
    pltpu.sync_copy(
        deg_sh.at[pl.ds(sid * ROWS_PER_TILE, ROWS_PER_TILE)],
        out_hbm.at[cid, pl.ds(sid * ROWS_PER_TILE, ROWS_PER_TILE)],
    )


@functools.partial(
    pl.kernel,
    out_type=jax.ShapeDtypeStruct((2, NPAD, F_B), jnp.float32),
    mesh=_mesh,
    compiler_params=_sc_params,
    scratch_types=[
        pltpu.VMEM((CH_PER_TILE + 1, CHUNK), jnp.int32),
        pltpu.VMEM((CH_PER_TILE, CHUNK), jnp.int32),
        pltpu.VMEM((CHUNK, F_B), jnp.float32),
        pltpu.VMEM((CHUNK, F_B), jnp.float32),
        pltpu.VMEM((CHUNK, F_B), jnp.float32),
        pltpu.VMEM((CHUNK, F_B), jnp.float32),
        pltpu.VMEM_SHARED((NPAD, F_B), jnp.float32),
        pltpu.SemaphoreType.DMA,
        pltpu.SemaphoreType.DMA,
        pltpu.SemaphoreType.DMA,
        pltpu.SemaphoreType.DMA,
    ],
)
def _sc_agg(b2_hbm, row_hbm, col_hbm, out_hbm, row_v, col_v,
            buf0, buf1, buf2, buf3, agg_sh, sem0, sem1, sem2, sem3):
    cid = lax.axis_index("c")
    sid = lax.axis_index("s")
    wid = sid * 2 + cid
    bufs = (buf0, buf1, buf2, buf3)
    sems = (sem0, sem1, sem2, sem3)
    NBUF = 4

    z16 = jnp.zeros((16,), jnp.float32)

    @pl.loop(0, CHUNK)
    def _(i):
        @pl.loop(0, F_B // 16)
        def _(k):
            buf0[i, pl.ds(k * 16, 16)] = z16

    @pl.loop(0, ROWS_PER_TILE // CHUNK)
    def _(k):
        pltpu.sync_copy(buf0, agg_sh.at[pl.ds(sid * ROWS_PER_TILE + k * CHUNK, CHUNK)])

    pltpu.sync_copy(row_hbm.at[pl.ds(wid * CH_PER_TILE, CH_PER_TILE)], row_v.at[pl.ds(0, CH_PER_TILE)])
    pltpu.sync_copy(col_hbm.at[pl.ds(wid * CH_PER_TILE, CH_PER_TILE)], col_v)
    z16i = jnp.zeros((16,), jnp.int32)
    for k in range(CHUNK // 16):
        row_v[CH_PER_TILE, pl.ds(k * 16, 16)] = z16i
    plsc.subcore_barrier()

    for b in range(NBUF):
        pltpu.async_copy(b2_hbm.at[row_v.at[b]], bufs[b], sems[b])

    @pl.loop(0, CH_PER_TILE // NBUF)
    def _(g):
        for b in range(NBUF):
            j = g * NBUF + b
            pltpu.make_async_copy(b2_hbm.at[row_v.at[j]], bufs[b], sems[b]).wait()
            jn = jnp.minimum(j + NBUF, CH_PER_TILE)
            pltpu.sync_copy(bufs[b], agg_sh.at[col_v.at[j]], add=True)
            pltpu.async_copy(b2_hbm.at[row_v.at[jn]], bufs[b], sems[b])

    for b in range(NBUF):
        pltpu.make_async_copy(b2_hbm.at[row_v.at[CH_PER_TILE]], bufs[b], sems[b]).wait()

    plsc.subcore_barrier()
    pltpu.sync_copy(
        agg_sh.at[pl.ds(sid * ROWS_PER_TILE, ROWS_PER_TILE)],
        out_hbm.at[cid, pl.ds(sid * ROWS_PER_TILE, ROWS_PER_TILE)],
    )


def _dense_body(x_ref, wb_ref, wc_ref, bc_ref, wr_ref, br_ref, b_ref, wt_ref, r_ref):
    xb = x_ref[...]
    b_ref[...] = jnp.dot(xb, wb_ref[...], preferred_element_type=jnp.float32)
    wt_ref[...] = jnp.dot(xb, wc_ref[...], preferred_element_type=jnp.float32) + bc_ref[...]
    r_ref[...] = jnp.dot(xb, wr_ref[...], preferred_element_type=jnp.float32) + br_ref[...]


_dense = pl.pallas_call(
    _dense_body,
    grid=(GRID,),
    in_specs=[
        pl.BlockSpec((BLK, 128), lambda i: (i, 0)),
        pl.BlockSpec((128, F_B), lambda i: (0, 0)),
        pl.BlockSpec((128, HEADS * BASES), lambda i: (0, 0)),
        pl.BlockSpec((1, HEADS * BASES), lambda i: (0, 0)),
        pl.BlockSpec((128, 128), lambda i: (0, 0)),
        pl.BlockSpec((1, 128), lambda i: (0, 0)),
    ],
    out_specs=[
        pl.BlockSpec((BLK, F_B), lambda i: (i, 0)),
        pl.BlockSpec((BLK, HEADS * BASES), lambda i: (i, 0)),
        pl.BlockSpec((BLK, 128), lambda i: (i, 0)),
    ],
    out_shape=[
        jax.ShapeDtypeStruct((NPAD, F_B), jnp.float32),
        jax.ShapeDtypeStruct((NPAD, HEADS * BASES), jnp.float32),
        jax.ShapeDtypeStruct((NPAD, 128), jnp.float32),
    ],
)


def _scale_body(d0_ref, d1_ref, bases_ref, b2_ref, dis_ref):
    deg = d0_ref[:, 0:1] + d1_ref[:, 0:1] + 1.0
    dis = lax.rsqrt(deg)
    dis_ref[...] = dis
    b2_ref[...] = bases_ref[...] * dis


_scale = pl.pallas_call(
    _scale_body,
    grid=(GRID,),
    in_specs=[
        pl.BlockSpec((BLK, 16), lambda i: (i, 0)),
        pl.BlockSpec((BLK, 16), lambda i: (i, 0)),
        pl.BlockSpec((BLK, F_B), lambda i: (i, 0)),
    ],
    out_specs=[
        pl.BlockSpec((BLK, F_B), lambda i: (i, 0)),
        pl.BlockSpec((BLK, 1), lambda i: (i, 0)),
    ],
    out_shape=[
        jax.ShapeDtypeStruct((NPAD, F_B), jnp.float32),
        jax.ShapeDtypeStruct((NPAD, 1), jnp.float32),
    ],
)


_P_np = np.zeros((BASES, HEADS * BASES, 128), np.float32)
_Q_np = np.zeros((BASES, F_B, 128), np.float32)
for _b in range(BASES):
    for _h in range(HEADS):
        for _f in range(F_H):
            _P_np[_b, _h * BASES + _b, _h * F_H + _f] = 1.0
            _Q_np[_b, _b * F_H + _f, _h * F_H + _f] = 1.0


def _finish_body(a0_ref, a1_ref, dis_ref, bases_ref, wt_ref, res_ref, bc_ref,
                 g_ref, bt_ref, p_ref, q_ref, o_ref):
    dis = dis_ref[...]
    aggf = dis * (a0_ref[...] + a1_ref[...]) + (dis * dis) * bases_ref[...]
    wt = wt_ref[...]
    conv = None
    for b in range(BASES):
        we = jnp.dot(wt, p_ref[b], preferred_element_type=jnp.float32)
        ae = jnp.dot(aggf, q_ref[b], preferred_element_type=jnp.float32)
        t = we * ae
        conv = t if conv is None else conv + t
    o = conv + bc_ref[...] + res_ref[...]
    mu = jnp.mean(o, axis=1, keepdims=True)
    var = jnp.mean((o - mu) * (o - mu), axis=1, keepdims=True)
    o = (o - mu) * lax.rsqrt(var + 1e-5) * g_ref[...] + bt_ref[...]
    o_ref[...] = jnp.maximum(o, 0.0)


_finish = pl.pallas_call(
    _finish_body,
    grid=(GRID,),
    in_specs=[
        pl.BlockSpec((BLK, F_B), lambda i: (i, 0)),
        pl.BlockSpec((BLK, F_B), lambda i: (i, 0)),
        pl.BlockSpec((BLK, 1), lambda i: (i, 0)),
        pl.BlockSpec((BLK, F_B), lambda i: (i, 0)),
        pl.BlockSpec((BLK, HEADS * BASES), lambda i: (i, 0)),
        pl.BlockSpec((BLK, 128), lambda i: (i, 0)),
        pl.BlockSpec((1, 128), lambda i: (0, 0)),
        pl.BlockSpec((1, 128), lambda i: (0, 0)),
        pl.BlockSpec((1, 128), lambda i: (0, 0)),
        pl.BlockSpec((BASES, HEADS * BASES, 128), lambda i: (0, 0, 0)),
        pl.BlockSpec((BASES, F_B, 128), lambda i: (0, 0, 0)),
    ],
    out_specs=pl.BlockSpec((BLK, 128), lambda i: (i, 0)),
    out_shape=jax.ShapeDtypeStruct((NPAD, 128), jnp.float32),
)


def kernel(x, edge_index, W_bases, W_comb, b_comb, bias_conv, W_res, b_res,
           ln_gamma, ln_beta):
    x_pad = jnp.zeros((NPAD, 128), jnp.float32).at[:N].set(x)
    row = edge_index[0]
    col = edge_index[1]
    pad = jnp.full((EPAD - E,), N, jnp.int32)
    row_p = jnp.concatenate([row, pad]).reshape(NCHUNKS, CHUNK)
    col_p = jnp.concatenate([col, pad]).reshape(NCHUNKS, CHUNK)

    bases, wt, res = _dense(x_pad, W_bases, W_comb, b_comb.reshape(1, -1),
                            W_res, b_res.reshape(1, -1))
    degp = _sc_degree(col_p)
    b2, dis = _scale(degp[0], degp[1], bases)
    aggp = _sc_agg(b2, row_p, col_p)
    out = _finish(aggp[0], aggp[1], dis, bases, wt, res,
                  bias_conv.reshape(1, -1), ln_gamma.reshape(1, -1),
                  ln_beta.reshape(1, -1), jnp.asarray(_P_np), jnp.asarray(_Q_np))
    return out[:N]


# speedup vs baseline: 14.3554x; 1.0044x over previous
import functools

import jax
import jax.numpy as jnp
import numpy as np
from jax import lax
from jax.experimental import pallas as pl
from jax.experimental.pallas import tpu as pltpu
from jax.experimental.pallas import tpu_sc as plsc

N = 10000
NPAD = 10240
E = 320000
CHUNK = 128
NCHUNKS = 2560
CH_PER_TILE = NCHUNKS // 32
EPAD = NCHUNKS * CHUNK
HEADS = 8
BASES = 4
F_H = 16
F_B = BASES * F_H
ROWS_PER_TILE = NPAD // 16
BLK = 256
GRID = NPAD // BLK

_mesh = plsc.VectorSubcoreMesh(core_axis_name="c", subcore_axis_name="s")
_sc_params = pltpu.CompilerParams(use_tc_tiling_on_sc=False)


@functools.partial(
    pl.kernel,
    out_type=jax.ShapeDtypeStruct((2, NPAD, 16), jnp.float32),
    mesh=_mesh,
    compiler_params=_sc_params,
    scratch_types=[
        pltpu.VMEM((CH_PER_TILE, CHUNK), jnp.int32),
        pltpu.VMEM((CHUNK, 16), jnp.float32),
        pltpu.VMEM_SHARED((NPAD, 16), jnp.float32),
        pltpu.SemaphoreType.DMA,
    ],
)
def _sc_degree(col_hbm, out_hbm, col_v, ones_v, deg_sh, sem):
    cid = lax.axis_index("c")
    sid = lax.axis_index("s")
    wid = sid * 2 + cid

    z16 = jnp.zeros((16,), jnp.float32)

    @pl.loop(0, CHUNK)
    def _(i):
        ones_v[i, pl.ds(0, 16)] = z16

    @pl.loop(0, ROWS_PER_TILE // CHUNK)
    def _(k):
        pltpu.sync_copy(ones_v, deg_sh.at[pl.ds(sid * ROWS_PER_TILE + k * CHUNK, CHUNK)])

    o16 = jnp.ones((16,), jnp.float32)

    @pl.loop(0, CHUNK)
    def _(i):
        ones_v[i, pl.ds(0, 16)] = o16

    pltpu.sync_copy(col_hbm.at[pl.ds(wid * CH_PER_TILE, CH_PER_TILE)], col_v)
    plsc.subcore_barrier()

    @pl.loop(0, CH_PER_TILE)
    def _(j):
        pltpu.async_copy(ones_v, deg_sh.at[col_v.at[j]], sem, add=True)

    @pl.loop(0, CH_PER_TILE)
    def _(j):
        pltpu.make_async_copy(ones_v, deg_sh.at[col_v.at[j]], sem).wait()

    plsc.subcore_barrier()
    pltpu.sync_copy(
        deg_sh.at[pl.ds(sid * ROWS_PER_TILE, ROWS_PER_TILE)],
        out_hbm.at[cid, pl.ds(sid * ROWS_PER_TILE, ROWS_PER_TILE)],
    )


@functools.partial(
    pl.kernel,
    out_type=jax.ShapeDtypeStruct((2, NPAD, F_B), jnp.float32),
    mesh=_mesh,
    compiler_params=_sc_params,
    scratch_types=[
        pltpu.VMEM((CH_PER_TILE + 1, CHUNK), jnp.int32),
        pltpu.VMEM((CH_PER_TILE, CHUNK), jnp.int32),
    ] + [pltpu.VMEM((CHUNK, F_B), jnp.float32)] * 8
      + [pltpu.VMEM_SHARED((NPAD, F_B), jnp.float32)]
      + [pltpu.SemaphoreType.DMA] * 16,
)
def _sc_agg(b2_hbm, row_hbm, col_hbm, out_hbm, row_v, col_v, *rest):
    bufs = rest[0:8]
    agg_sh = rest[8]
    sem_g = rest[9:17]
    sem_s = rest[17:25]
    cid = lax.axis_index("c")
    sid = lax.axis_index("s")
    wid = sid * 2 + cid
    NBUF = 8
    LAG = 4

    z16 = jnp.zeros((16,), jnp.float32)

    @pl.loop(0, CHUNK)
    def _(i):
        @pl.loop(0, F_B // 16)
        def _(k):
            bufs[0][i, pl.ds(k * 16, 16)] = z16

    @pl.loop(0, ROWS_PER_TILE // CHUNK)
    def _(k):
        pltpu.sync_copy(bufs[0], agg_sh.at[pl.ds(sid * ROWS_PER_TILE + k * CHUNK, CHUNK)])

    pltpu.sync_copy(row_hbm.at[pl.ds(wid * CH_PER_TILE, CH_PER_TILE)], row_v.at[pl.ds(0, CH_PER_TILE)])
    pltpu.sync_copy(col_hbm.at[pl.ds(wid * CH_PER_TILE, CH_PER_TILE)], col_v)
    z16i = jnp.zeros((16,), jnp.int32)
    for k in range(CHUNK // 16):
        row_v[CH_PER_TILE, pl.ds(k * 16, 16)] = z16i
    plsc.subcore_barrier()

    def _gather(chunk_idx, b, clamp=False):
        ci = jnp.minimum(chunk_idx, CH_PER_TILE) if clamp else chunk_idx
        pltpu.async_copy(b2_hbm.at[row_v.at[ci]], bufs[b], sem_g[b])

    for b in range(LAG):
        _gather(b, b)

    @pl.loop(0, CH_PER_TILE // NBUF)
    def _(g):
        for b in range(NBUF):
            i = g * NBUF + b
            bg = (b + LAG) % NBUF
            if b < LAG:
                @pl.when(g > 0)
                def _():
                    pltpu.make_async_copy(bufs[bg], agg_sh.at[col_v.at[i]], sem_s[bg]).wait()
            else:
                pltpu.make_async_copy(bufs[bg], agg_sh.at[col_v.at[i]], sem_s[bg]).wait()
            _gather(i + LAG, bg, clamp=True)
            pltpu.make_async_copy(b2_hbm.at[row_v.at[i]], bufs[b], sem_g[b]).wait()
            pltpu.async_copy(bufs[b], agg_sh.at[col_v.at[i]], sem_s[b], add=True)

    for b in range(LAG, NBUF):
        pltpu.make_async_copy(bufs[b], agg_sh.at[col_v.at[0]], sem_s[b]).wait()
    for b in range(LAG):
        pltpu.make_async_copy(b2_hbm.at[row_v.at[CH_PER_TILE]], bufs[b], sem_g[b]).wait()

    plsc.subcore_barrier()
    pltpu.sync_copy(
        agg_sh.at[pl.ds(sid * ROWS_PER_TILE, ROWS_PER_TILE)],
        out_hbm.at[cid, pl.ds(sid * ROWS_PER_TILE, ROWS_PER_TILE)],
    )


def _dense_body(x_ref, wb_ref, wc_ref, bc_ref, wr_ref, br_ref, b_ref, wt_ref, r_ref):
    xb = x_ref[...]
    b_ref[...] = jnp.dot(xb, wb_ref[...], preferred_element_type=jnp.float32)
    wt_ref[...] = jnp.dot(xb, wc_ref[...], preferred_element_type=jnp.float32) + bc_ref[...]
    r_ref[...] = jnp.dot(xb, wr_ref[...], preferred_element_type=jnp.float32) + br_ref[...]


_dense = pl.pallas_call(
    _dense_body,
    grid=(GRID,),
    in_specs=[
        pl.BlockSpec((BLK, 128), lambda i: (i, 0)),
        pl.BlockSpec((128, F_B), lambda i: (0, 0)),
        pl.BlockSpec((128, HEADS * BASES), lambda i: (0, 0)),
        pl.BlockSpec((1, HEADS * BASES), lambda i: (0, 0)),
        pl.BlockSpec((128, 128), lambda i: (0, 0)),
        pl.BlockSpec((1, 128), lambda i: (0, 0)),
    ],
    out_specs=[
        pl.BlockSpec((BLK, F_B), lambda i: (i, 0)),
        pl.BlockSpec((BLK, HEADS * BASES), lambda i: (i, 0)),
        pl.BlockSpec((BLK, 128), lambda i: (i, 0)),
    ],
    out_shape=[
        jax.ShapeDtypeStruct((NPAD, F_B), jnp.float32),
        jax.ShapeDtypeStruct((NPAD, HEADS * BASES), jnp.float32),
        jax.ShapeDtypeStruct((NPAD, 128), jnp.float32),
    ],
)


def _scale_body(d0_ref, d1_ref, bases_ref, b2_ref, dis_ref):
    deg = d0_ref[:, 0:1] + d1_ref[:, 0:1] + 1.0
    dis = lax.rsqrt(deg)
    dis_ref[...] = dis
    b2_ref[...] = bases_ref[...] * dis


_scale = pl.pallas_call(
    _scale_body,
    grid=(GRID,),
    in_specs=[
        pl.BlockSpec((BLK, 16), lambda i: (i, 0)),
        pl.BlockSpec((BLK, 16), lambda i: (i, 0)),
        pl.BlockSpec((BLK, F_B), lambda i: (i, 0)),
    ],
    out_specs=[
        pl.BlockSpec((BLK, F_B), lambda i: (i, 0)),
        pl.BlockSpec((BLK, 1), lambda i: (i, 0)),
    ],
    out_shape=[
        jax.ShapeDtypeStruct((NPAD, F_B), jnp.float32),
        jax.ShapeDtypeStruct((NPAD, 1), jnp.float32),
    ],
)


_P_np = np.zeros((BASES, HEADS * BASES, 128), np.float32)
_Q_np = np.zeros((BASES, F_B, 128), np.float32)
for _b in range(BASES):
    for _h in range(HEADS):
        for _f in range(F_H):
            _P_np[_b, _h * BASES + _b, _h * F_H + _f] = 1.0
            _Q_np[_b, _b * F_H + _f, _h * F_H + _f] = 1.0


def _finish_body(a0_ref, a1_ref, dis_ref, bases_ref, wt_ref, res_ref, bc_ref,
                 g_ref, bt_ref, p_ref, q_ref, o_ref):
    dis = dis_ref[...]
    aggf = dis * (a0_ref[...] + a1_ref[...]) + (dis * dis) * bases_ref[...]
    wt = wt_ref[...]
    conv = None
    for b in range(BASES):
        we = jnp.dot(wt, p_ref[b], preferred_element_type=jnp.float32)
        ae = jnp.dot(aggf, q_ref[b], preferred_element_type=jnp.float32)
        t = we * ae
        conv = t if conv is None else conv + t
    o = conv + bc_ref[...] + res_ref[...]
    mu = jnp.mean(o, axis=1, keepdims=True)
    var = jnp.mean((o - mu) * (o - mu), axis=1, keepdims=True)
    o = (o - mu) * lax.rsqrt(var + 1e-5) * g_ref[...] + bt_ref[...]
    o_ref[...] = jnp.maximum(o, 0.0)


_finish = pl.pallas_call(
    _finish_body,
    grid=(GRID,),
    in_specs=[
        pl.BlockSpec((BLK, F_B), lambda i: (i, 0)),
        pl.BlockSpec((BLK, F_B), lambda i: (i, 0)),
        pl.BlockSpec((BLK, 1), lambda i: (i, 0)),
        pl.BlockSpec((BLK, F_B), lambda i: (i, 0)),
        pl.BlockSpec((BLK, HEADS * BASES), lambda i: (i, 0)),
        pl.BlockSpec((BLK, 128), lambda i: (i, 0)),
        pl.BlockSpec((1, 128), lambda i: (0, 0)),
        pl.BlockSpec((1, 128), lambda i: (0, 0)),
        pl.BlockSpec((1, 128), lambda i: (0, 0)),
        pl.BlockSpec((BASES, HEADS * BASES, 128), lambda i: (0, 0, 0)),
        pl.BlockSpec((BASES, F_B, 128), lambda i: (0, 0, 0)),
    ],
    out_specs=pl.BlockSpec((BLK, 128), lambda i: (i, 0)),
    out_shape=jax.ShapeDtypeStruct((NPAD, 128), jnp.float32),
)


def kernel(x, edge_index, W_bases, W_comb, b_comb, bias_conv, W_res, b_res,
           ln_gamma, ln_beta):
    x_pad = jnp.zeros((NPAD, 128), jnp.float32).at[:N].set(x)
    row = edge_index[0]
    col = edge_index[1]
    pad = jnp.full((EPAD - E,), N, jnp.int32)
    row_p = jnp.concatenate([row, pad]).reshape(NCHUNKS, CHUNK)
    col_p = jnp.concatenate([col, pad]).reshape(NCHUNKS, CHUNK)

    bases, wt, res = _dense(x_pad, W_bases, W_comb, b_comb.reshape(1, -1),
                            W_res, b_res.reshape(1, -1))
    degp = _sc_degree(col_p)
    b2, dis = _scale(degp[0], degp[1], bases)
    aggp = _sc_agg(b2, row_p, col_p)
    out = _finish(aggp[0], aggp[1], dis, bases, wt, res,
                  bias_conv.reshape(1, -1), ln_gamma.reshape(1, -1),
                  ln_beta.reshape(1, -1), jnp.asarray(_P_np), jnp.asarray(_Q_np))
    return out[:N]
